# Initial kernel scaffold; baseline (speedup 1.0000x reference)
import jax
import jax.numpy as jnp
from jax.experimental import pallas as pl


def kernel(x, edge_index, batch, W1, b1, g1, be1, W2, b2, g2, be2, W3, b3, g3, be3, W4, b4):
    raise NotImplementedError("write your pallas kernel here")


# speedup vs baseline: 9.8174x; 9.8174x over previous
import functools

import jax
import jax.numpy as jnp
from jax import lax
from jax.experimental import pallas as pl
from jax.experimental.pallas import tpu as pltpu
from jax.experimental.pallas import tpu_sc as plsc

_N = 10000
_D = 128
_H = 256
_C = 10
_G = 64
_E = 320000

_NC = 2
_NS = 16
_NW = _NC * _NS

_CHUNK = 128
_CPT = -(-_E // (_NW * _CHUNK))
_EPAD = _NW * _CPT * _CHUNK
_RPT = 640
_NP = _NS * _RPT

@functools.cache
def _sc_kernels():
    mesh = plsc.VectorSubcoreMesh(
        core_axis_name="c", subcore_axis_name="s", num_cores=_NC, num_subcores=_NS
    )

    @functools.partial(
        pl.kernel,
        out_type=jax.ShapeDtypeStruct((_NC, _NP, _D), jnp.float32),
        mesh=mesh,
        scratch_types=[
            pltpu.VMEM((_CHUNK,), jnp.int32),
            pltpu.VMEM((_CHUNK, _D), jnp.float32),
            pltpu.VMEM_SHARED((_NP, _D), jnp.float32),
        ],
    )
    def _deg(dstp_hbm, onerows_hbm, zblk_hbm, out_hbm, dst_v, ones_v, sh):
        c = lax.axis_index("c")
        s = lax.axis_index("s")
        wid = s * _NC + c
        pltpu.sync_copy(onerows_hbm, ones_v)

        def zloop(i, carry):
            pltpu.sync_copy(zblk_hbm, sh.at[pl.ds(s * _RPT + i * _CHUNK, _CHUNK), :])
            return carry

        lax.fori_loop(0, _RPT // _CHUNK, zloop, 0)
        plsc.subcore_barrier()
        base = wid * (_CPT * _CHUNK)

        def eloop(ci, carry):
            pltpu.sync_copy(dstp_hbm.at[pl.ds(base + ci * _CHUNK, _CHUNK)], dst_v)
            pltpu.sync_copy(ones_v, sh.at[dst_v], add=True)
            return carry

        lax.fori_loop(0, _CPT, eloop, 0)
        plsc.subcore_barrier()

        def oloop(i, carry):
            r0 = s * _RPT + i * _CHUNK
            pltpu.sync_copy(sh.at[pl.ds(r0, _CHUNK), :], out_hbm.at[c, pl.ds(r0, _CHUNK), :])
            return carry

        lax.fori_loop(0, _RPT // _CHUNK, oloop, 0)

    @functools.partial(
        pl.kernel,
        out_type=jax.ShapeDtypeStruct((_NC, _NP, _D), jnp.float32),
        mesh=mesh,
        scratch_types=[
            pltpu.VMEM((_CHUNK,), jnp.int32),
            pltpu.VMEM((_CHUNK,), jnp.int32),
            pltpu.VMEM((_CHUNK, _D), jnp.float32),
            pltpu.VMEM_SHARED((_NP, _D), jnp.float32),
            pltpu.SemaphoreType.DMA,
        ],
    )
    def _agg(hs_hbm, srcp_hbm, dstp_hbm, zblk_hbm, out_hbm, src_v, dst_v, rows_v, acc_sh, sem):
        c = lax.axis_index("c")
        s = lax.axis_index("s")
        wid = s * _NC + c

        def zloop(i, carry):
            pltpu.sync_copy(zblk_hbm, acc_sh.at[pl.ds(s * _RPT + i * _CHUNK, _CHUNK), :])
            return carry

        lax.fori_loop(0, _RPT // _CHUNK, zloop, 0)
        plsc.subcore_barrier()

        base = wid * (_CPT * _CHUNK)

        def eloop(ci, carry):
            off = base + ci * _CHUNK
            pltpu.sync_copy(srcp_hbm.at[pl.ds(off, _CHUNK)], src_v)
            pltpu.async_copy(hs_hbm.at[src_v], rows_v, sem).wait()
            pltpu.sync_copy(dstp_hbm.at[pl.ds(off, _CHUNK)], dst_v)
            pltpu.sync_copy(rows_v, acc_sh.at[dst_v], add=True)
            return carry

        lax.fori_loop(0, _CPT, eloop, 0)
        plsc.subcore_barrier()

        def oloop(i, carry):
            r0 = s * _RPT + i * _CHUNK
            pltpu.sync_copy(acc_sh.at[pl.ds(r0, _CHUNK), :], out_hbm.at[c, pl.ds(r0, _CHUNK), :])
            return carry

        lax.fori_loop(0, _RPT // _CHUNK, oloop, 0)

    return _deg, _agg


def _tc1_body(x_ref, w_ref, dg_ref, hs_ref, dinv_ref):
    deg = dg_ref[0, :, :] + dg_ref[1, :, :] + 1.0
    dinv = lax.rsqrt(deg)
    h = jnp.dot(x_ref[...], w_ref[...], preferred_element_type=jnp.float32)
    hs_ref[...] = h * dinv
    dinv_ref[...] = dinv


def _tc1(x, w1, deg2):
    return pl.pallas_call(
        _tc1_body,
        out_shape=(
            jax.ShapeDtypeStruct((_N, _D), jnp.float32),
            jax.ShapeDtypeStruct((_N, 1), jnp.float32),
        ),
    )(x, w1, deg2)


def _bn_relu(t, g, be):
    mu = jnp.mean(t, axis=0, keepdims=True)
    var = jnp.mean((t - mu) ** 2, axis=0, keepdims=True)
    return jnp.maximum((t - mu) * lax.rsqrt(var + 1e-5) * g + be, 0.0)


def _tcmid_body(a_ref, hs_ref, dinv_ref, b_ref, g_ref, be_ref, w_ref, hs2_ref):
    p = a_ref[0, : _N, :] + a_ref[1, : _N, :]
    dinv = dinv_ref[...]
    t = (p + hs_ref[...]) * dinv + b_ref[...]
    h = _bn_relu(t, g_ref[...], be_ref[...])
    hs2_ref[...] = jnp.dot(h, w_ref[...], preferred_element_type=jnp.float32) * dinv


def _tcmid(agg, hs, dinv, b, g, be, w):
    return pl.pallas_call(
        _tcmid_body,
        out_shape=jax.ShapeDtypeStruct((_N, _D), jnp.float32),
    )(agg, hs, dinv, b, g, be, w)


def _tcfin_body(a_ref, hs_ref, dinv_ref, b_ref, g_ref, be_ref, batch_ref,
                w3_ref, b3_ref, g3_ref, be3_ref, w4_ref, b4_ref, out_ref):
    p = a_ref[0, : _N, :] + a_ref[1, : _N, :]
    t = (p + hs_ref[...]) * dinv_ref[...] + b_ref[...]
    h = _bn_relu(t, g_ref[...], be_ref[...])
    m = (batch_ref[...] == lax.broadcasted_iota(jnp.int32, (_N, _G), 1)).astype(jnp.float32)
    sums = lax.dot_general(m, h, (((0,), (0,)), ((), ())),
                           preferred_element_type=jnp.float32)
    cnts = jnp.sum(m, axis=0)[:, None]
    pooled = sums / jnp.maximum(cnts, 1.0)
    z = jnp.dot(pooled, w3_ref[...], preferred_element_type=jnp.float32) + b3_ref[...]
    z = _bn_relu(z, g3_ref[...], be3_ref[...])
    out_ref[...] = jnp.dot(z, w4_ref[...], preferred_element_type=jnp.float32) + b4_ref[...]


def _tcfin(agg, hs, dinv, b, g, be, batch2d, w3, b3, g3, be3, w4, b4):
    return pl.pallas_call(
        _tcfin_body,
        out_shape=jax.ShapeDtypeStruct((_G, _C), jnp.float32),
    )(agg, hs, dinv, b, g, be, batch2d, w3, b3, g3, be3, w4, b4)


def kernel(x, edge_index, batch, W1, b1, g1, be1, W2, b2, g2, be2,
           W3, b3, g3, be3, W4, b4):
    src = edge_index[0]
    dst = edge_index[1]
    npad = _EPAD - _E
    srcp = jnp.concatenate([src, jnp.zeros((npad,), jnp.int32)])
    dstp = jnp.concatenate([dst, jnp.full((npad,), _N, jnp.int32)])
    zblk = jnp.zeros((_CHUNK, _D), jnp.float32)
    onerows = jnp.ones((_CHUNK, _D), jnp.float32)

    _deg, _agg = _sc_kernels()
    deg2 = _deg(dstp, onerows, zblk)[:, : _N, 0:1]

    hs1, dinv = _tc1(x, W1, deg2)
    agg1 = _agg(hs1, srcp, dstp, zblk)
    hs2 = _tcmid(agg1, hs1, dinv, b1[None, :], g1[None, :], be1[None, :], W2)
    agg2 = _agg(hs2, srcp, dstp, zblk)
    out = _tcfin(agg2, hs2, dinv, b2[None, :], g2[None, :], be2[None, :],
                 batch[:, None], W3, b3[None, :], g3[None, :], be3[None, :],
                 W4, b4[None, :])
    return out


# speedup vs baseline: 21.6234x; 2.2026x over previous
import functools

import jax
import jax.numpy as jnp
from jax import lax
from jax.experimental import pallas as pl
from jax.experimental.pallas import tpu as pltpu
from jax.experimental.pallas import tpu_sc as plsc

_N = 10000
_D = 128
_H = 256
_C = 10
_G = 64
_E = 320000

_NC = 2
_NS = 16
_NW = _NC * _NS

_CHUNK = 128
_CPT = 80
_CPH = 40
_EPAD = _NW * _CPT * _CHUNK
_RPT = 640
_NP = _NS * _RPT

@functools.cache
def _sc_kernels():
    mesh = plsc.VectorSubcoreMesh(
        core_axis_name="c", subcore_axis_name="s", num_cores=_NC, num_subcores=_NS
    )

    @functools.partial(
        pl.kernel,
        out_type=jax.ShapeDtypeStruct((_NC, _NP, _D), jnp.float32),
        mesh=mesh,
        scratch_types=[
            pltpu.VMEM((_CPT, _CHUNK), jnp.int32),
            pltpu.VMEM((_CHUNK, _D), jnp.float32),
            pltpu.VMEM_SHARED((_NP, _D), jnp.float32),
            pltpu.SemaphoreType.DMA,
        ],
    )
    def _deg(dstp_hbm, onerows_hbm, zblk_hbm, out_hbm, dst_v, ones_v, sh, sem):
        c = lax.axis_index("c")
        s = lax.axis_index("s")
        wid = s * _NC + c
        pltpu.sync_copy(onerows_hbm, ones_v)
        pltpu.sync_copy(dstp_hbm.at[wid], dst_v)

        def zloop(i, carry):
            pltpu.sync_copy(zblk_hbm, sh.at[pl.ds(s * _RPT + i * _CHUNK, _CHUNK), :])
            return carry

        lax.fori_loop(0, _RPT // _CHUNK, zloop, 0)
        plsc.subcore_barrier()

        def eloop(ci, carry):
            pltpu.async_copy(ones_v, sh.at[dst_v.at[ci]], sem, add=True)
            return carry

        lax.fori_loop(0, _CPT, eloop, 0)

        def dloop(ci, carry):
            pltpu.make_async_copy(ones_v, sh.at[dst_v.at[0]], sem).wait()
            return carry

        lax.fori_loop(0, _CPT, dloop, 0)
        plsc.subcore_barrier()

        def oloop(i, carry):
            r0 = s * _RPT + i * _CHUNK
            pltpu.sync_copy(sh.at[pl.ds(r0, _CHUNK), :], out_hbm.at[c, pl.ds(r0, _CHUNK), :])
            return carry

        lax.fori_loop(0, _RPT // _CHUNK, oloop, 0)

    @functools.partial(
        pl.kernel,
        out_type=jax.ShapeDtypeStruct((_NC, _NP, _D), jnp.float32),
        mesh=mesh,
        scratch_types=[
            pltpu.VMEM((_CPH, _CHUNK), jnp.int32),
            pltpu.VMEM((_CPH, _CHUNK), jnp.int32),
            pltpu.VMEM((_CHUNK, _D), jnp.float32),
            pltpu.VMEM((_CHUNK, _D), jnp.float32),
            pltpu.VMEM_SHARED((_NP, _D), jnp.float32),
            pltpu.SemaphoreType.DMA,
            pltpu.SemaphoreType.DMA,
            pltpu.SemaphoreType.DMA,
            pltpu.SemaphoreType.DMA,
        ],
    )
    def _agg(hs_hbm, srcp_hbm, dstp_hbm, zblk_hbm, out_hbm,
             src_v, dst_v, rows0, rows1, acc_sh, g0, g1, s0, s1):
        c = lax.axis_index("c")
        s = lax.axis_index("s")
        wid = s * _NC + c
        rows = (rows0, rows1)

        def zloop(i, carry):
            pltpu.sync_copy(zblk_hbm, acc_sh.at[pl.ds(s * _RPT + i * _CHUNK, _CHUNK), :])
            return carry

        lax.fori_loop(0, _RPT // _CHUNK, zloop, 0)
        plsc.subcore_barrier()

        def drain_scatter(p):
            pltpu.make_async_copy(rows[p], acc_sh.at[dst_v.at[0]], (s0, s1)[p]).wait()

        def pair(g, drain_prev):
            i0 = 2 * g
            i1 = i0 + 1
            if drain_prev:
                drain_scatter(0)
            gd0 = pltpu.async_copy(hs_hbm.at[src_v.at[i0]], rows0, g0)
            if drain_prev:
                drain_scatter(1)
            gd1 = pltpu.async_copy(hs_hbm.at[src_v.at[i1]], rows1, g1)
            gd0.wait()
            pltpu.async_copy(rows0, acc_sh.at[dst_v.at[i0]], s0, add=True)
            gd1.wait()
            pltpu.async_copy(rows1, acc_sh.at[dst_v.at[i1]], s1, add=True)

        for h in range(_CPT // _CPH):
            if h > 0:
                drain_scatter(0)
                drain_scatter(1)
            pltpu.sync_copy(srcp_hbm.at[wid, pl.ds(h * _CPH, _CPH)], src_v)
            pltpu.sync_copy(dstp_hbm.at[wid, pl.ds(h * _CPH, _CPH)], dst_v)
            pair(0, False)

            def eloop(g, carry):
                pair(g, True)
                return carry

            lax.fori_loop(1, _CPH // 2, eloop, 0)

        drain_scatter(0)
        drain_scatter(1)
        plsc.subcore_barrier()

        def oloop(i, carry):
            r0 = s * _RPT + i * _CHUNK
            pltpu.sync_copy(acc_sh.at[pl.ds(r0, _CHUNK), :], out_hbm.at[c, pl.ds(r0, _CHUNK), :])
            return carry

        lax.fori_loop(0, _RPT // _CHUNK, oloop, 0)

    return _deg, _agg


def _tc1_body(x_ref, w_ref, dg_ref, hs_ref, dinv_ref):
    deg = dg_ref[0, :, :] + dg_ref[1, :, :] + 1.0
    dinv = lax.rsqrt(deg)
    h = jnp.dot(x_ref[...], w_ref[...], preferred_element_type=jnp.float32)
    hs_ref[...] = h * dinv
    dinv_ref[...] = dinv


def _tc1(x, w1, deg2):
    return pl.pallas_call(
        _tc1_body,
        out_shape=(
            jax.ShapeDtypeStruct((_N, _D), jnp.float32),
            jax.ShapeDtypeStruct((_N, 1), jnp.float32),
        ),
    )(x, w1, deg2)


def _bn_relu(t, g, be):
    mu = jnp.mean(t, axis=0, keepdims=True)
    var = jnp.mean((t - mu) ** 2, axis=0, keepdims=True)
    return jnp.maximum((t - mu) * lax.rsqrt(var + 1e-5) * g + be, 0.0)


def _tcmid_body(a_ref, hs_ref, dinv_ref, b_ref, g_ref, be_ref, w_ref, hs2_ref):
    p = a_ref[0, : _N, :] + a_ref[1, : _N, :]
    dinv = dinv_ref[...]
    t = (p + hs_ref[...]) * dinv + b_ref[...]
    h = _bn_relu(t, g_ref[...], be_ref[...])
    hs2_ref[...] = jnp.dot(h, w_ref[...], preferred_element_type=jnp.float32) * dinv


def _tcmid(agg, hs, dinv, b, g, be, w):
    return pl.pallas_call(
        _tcmid_body,
        out_shape=jax.ShapeDtypeStruct((_N, _D), jnp.float32),
    )(agg, hs, dinv, b, g, be, w)


def _tcfin_body(a_ref, hs_ref, dinv_ref, b_ref, g_ref, be_ref, batch_ref,
                w3_ref, b3_ref, g3_ref, be3_ref, w4_ref, b4_ref, out_ref):
    p = a_ref[0, : _N, :] + a_ref[1, : _N, :]
    t = (p + hs_ref[...]) * dinv_ref[...] + b_ref[...]
    h = _bn_relu(t, g_ref[...], be_ref[...])
    m = (batch_ref[...] == lax.broadcasted_iota(jnp.int32, (_N, _G), 1)).astype(jnp.float32)
    sums = lax.dot_general(m, h, (((0,), (0,)), ((), ())),
                           preferred_element_type=jnp.float32)
    cnts = jnp.sum(m, axis=0)[:, None]
    pooled = sums / jnp.maximum(cnts, 1.0)
    z = jnp.dot(pooled, w3_ref[...], preferred_element_type=jnp.float32) + b3_ref[...]
    z = _bn_relu(z, g3_ref[...], be3_ref[...])
    out_ref[...] = jnp.dot(z, w4_ref[...], preferred_element_type=jnp.float32) + b4_ref[...]


def _tcfin(agg, hs, dinv, b, g, be, batch2d, w3, b3, g3, be3, w4, b4):
    return pl.pallas_call(
        _tcfin_body,
        out_shape=jax.ShapeDtypeStruct((_G, _C), jnp.float32),
    )(agg, hs, dinv, b, g, be, batch2d, w3, b3, g3, be3, w4, b4)


def kernel(x, edge_index, batch, W1, b1, g1, be1, W2, b2, g2, be2,
           W3, b3, g3, be3, W4, b4):
    src = edge_index[0]
    dst = edge_index[1]
    npad = _EPAD - _E
    pad = jnp.arange(npad, dtype=jnp.int32)
    srcp = jnp.concatenate([src, pad % _N]).reshape(_NW, _CPT, _CHUNK)
    dstp = jnp.concatenate([dst, _N + pad % (_NP - _N)]).reshape(_NW, _CPT, _CHUNK)
    zblk = jnp.zeros((_CHUNK, _D), jnp.float32)
    onerows = jnp.ones((_CHUNK, _D), jnp.float32)

    _deg, _agg = _sc_kernels()
    deg2 = _deg(dstp, onerows, zblk)[:, : _N, 0:1]

    hs1, dinv = _tc1(x, W1, deg2)
    agg1 = _agg(hs1, srcp, dstp, zblk)
    hs2 = _tcmid(agg1, hs1, dinv, b1[None, :], g1[None, :], be1[None, :], W2)
    agg2 = _agg(hs2, srcp, dstp, zblk)
    out = _tcfin(agg2, hs2, dinv, b2[None, :], g2[None, :], be2[None, :],
                 batch[:, None], W3, b3[None, :], g3[None, :], be3[None, :],
                 W4, b4[None, :])
    return out
